# baseline (speedup 1.0000x reference)
import jax
import jax.numpy as jnp
from jax.experimental import pallas as pl
from jax.experimental.pallas import tpu as pltpu

_M = 256
_ITERS = 8
_TOL = 1e-4
_K = 15
_NEG = -1e9
_CH = 4096


def _body(points_ref, features_ref, centroids_ref, rpts_ref, rfeats_ref,
          cent_s, work_s):
    b = pl.program_id(0)

    @pl.when(b == 0)
    def _():
        cent_s[...] = centroids_ref[...]

    xT = points_ref[0]
    n = xT.shape[1]
    nch = n // _CH
    riota = jax.lax.broadcasted_iota(jnp.int32, (_M, _CH), 0)
    ciota = jax.lax.broadcasted_iota(jnp.int32, (_M, _CH), 1)
    x2 = jnp.sum(xT * xT, axis=0, keepdims=True)

    def sim_chunk(cent, c2, c):
        sl = slice(c * _CH, (c + 1) * _CH)
        xc = xT[:, sl]
        cx = jax.lax.dot_general(cent, xc, (((1,), (0,)), ((), ())),
                                 preferred_element_type=jnp.float32)
        return 2.0 * cx - c2 - x2[:, sl], xc

    x_ext = jnp.concatenate([xT, jnp.ones((1, n), jnp.float32)], axis=0)

    def km_iter(_, st):
        cent, num_pts, closest, done = st
        c2h = 0.5 * jnp.sum(cent * cent, axis=1, keepdims=True)
        acc = jnp.zeros((_M, 4), jnp.float32)
        parts = []
        for c in range(nch):
            xc = xT[:, c * _CH:(c + 1) * _CH]
            simc = jax.lax.dot_general(
                cent, xc, (((1,), (0,)), ((), ())),
                preferred_element_type=jnp.float32) - c2h
            closc = jnp.argmax(simc, axis=0).reshape(1, _CH)
            maskc = (riota == closc).astype(jnp.float32)
            acc = acc + jax.lax.dot_general(
                maskc, x_ext[:, c * _CH:(c + 1) * _CH],
                (((1,), (1,)), ((), ())),
                preferred_element_type=jnp.float32)
            parts.append(closc)
        new_closest = jnp.concatenate(parts, axis=1)
        counts = acc[:, 3:4]
        c_grad = acc[:, 0:3] / (counts + 1e-8)
        error = jnp.sum((c_grad - cent) ** 2)
        lr = 0.9 / (num_pts + 1e-8) + 0.1
        cent_n = cent * (1.0 - lr) + c_grad * lr
        num_n = num_pts + counts
        cent = jnp.where(done, cent, cent_n)
        num_pts = jnp.where(done, num_pts, num_n)
        closest = jnp.where(done, closest, new_closest)
        done = jnp.logical_or(done, error <= _TOL)
        return cent, num_pts, closest, done

    st0 = (cent_s[...], jnp.ones((_M, 1), jnp.float32),
           jnp.zeros((1, n), jnp.int32), jnp.zeros((), jnp.bool_))
    cent, _, closest, _ = jax.lax.fori_loop(0, _ITERS, km_iter, st0,
                                            unroll=False)
    cent_s[...] = cent
    rpts_ref[0] = cent

    c2 = jnp.sum(cent * cent, axis=1, keepdims=True)
    cnt = jnp.zeros((_M, 1), jnp.float32)
    cands = []
    for c in range(nch):
        sl = slice(c * _CH, (c + 1) * _CH)
        simc, _ = sim_chunk(cent, c2, c)
        member = riota == closest[:, sl]
        wc = jnp.where(member, simc, _NEG)
        work_s[:, sl] = wc
        cnt = cnt + jnp.sum(member.astype(jnp.float32), axis=1,
                            keepdims=True)
        v = jnp.max(wc, axis=1, keepdims=True)
        cands.append(v)
        for _ in range(_K - 1):
            v = jnp.max(jnp.where(wc < v, wc, -jnp.inf), axis=1,
                        keepdims=True)
            cands.append(v)
    cand = jnp.concatenate(cands, axis=1)
    candiota = jax.lax.broadcasted_iota(jnp.int32, cand.shape, 1)
    m0 = jnp.max(cand, axis=1, keepdims=True)
    t = m0
    for _ in range(_K - 1):
        am = jnp.argmax(cand, axis=1).reshape(_M, 1)
        cand = jnp.where(candiota == am, -jnp.inf, cand)
        t = jnp.max(cand, axis=1, keepdims=True)

    empty = cnt == 0.0
    total = jnp.zeros((_M, 1), jnp.float32)
    cf = jnp.zeros((_M, 64), jnp.float32)
    for c in range(nch):
        sl = slice(c * _CH, (c + 1) * _CH)
        wc = work_s[:, sl]
        w = jnp.where(wc >= t, jnp.exp(wc - m0), 0.0)
        if c == 0:
            w = jnp.where(empty, (ciota < _K).astype(jnp.float32), w)
        else:
            w = jnp.where(empty, 0.0, w)
        total = total + jnp.sum(w, axis=1, keepdims=True)
        cf = cf + jax.lax.dot_general(
            w, features_ref[0, :, sl], (((1,), (1,)), ((), ())),
            preferred_element_type=jnp.float32)
    rfeats_ref[0] = cf / total


@jax.jit
def kernel(points, features, centroids):
    nb, n, _ = points.shape
    nf = features.shape[1]
    points_t = jnp.swapaxes(points, 1, 2)
    rpts, rfeats_t = pl.pallas_call(
        _body,
        grid=(nb,),
        in_specs=[
            pl.BlockSpec((1, 3, n), lambda b: (b, 0, 0)),
            pl.BlockSpec((1, nf, n), lambda b: (b, 0, 0)),
            pl.BlockSpec((_M, 3), lambda b: (0, 0)),
        ],
        out_specs=[
            pl.BlockSpec((1, _M, 3), lambda b: (b, 0, 0)),
            pl.BlockSpec((1, _M, nf), lambda b: (b, 0, 0)),
        ],
        out_shape=[
            jax.ShapeDtypeStruct((nb, _M, 3), jnp.float32),
            jax.ShapeDtypeStruct((nb, _M, nf), jnp.float32),
        ],
        scratch_shapes=[
            pltpu.VMEM((_M, 3), jnp.float32),
            pltpu.VMEM((_M, n), jnp.float32),
        ],
        compiler_params=pltpu.CompilerParams(
            dimension_semantics=("arbitrary",)),
    )(points_t, features, centroids)
    r_feats = jnp.swapaxes(rfeats_t, 1, 2)
    return rpts, r_feats, rpts[nb - 1]


# speedup vs baseline: 1.0764x; 1.0764x over previous
import jax
import jax.numpy as jnp
from jax.experimental import pallas as pl
from jax.experimental.pallas import tpu as pltpu

_M = 256
_ITERS = 8
_TOL = 1e-4
_K = 15
_NEG = -1e9
_CH = 2048


def _body(points_ref, features_ref, centroids_ref, rpts_ref, rfeats_ref,
          cent_s, work_s):
    b = pl.program_id(0)

    @pl.when(b == 0)
    def _():
        cent_s[...] = centroids_ref[...]

    xT = points_ref[0]
    n = xT.shape[1]
    nch = n // _CH
    riota = jax.lax.broadcasted_iota(jnp.int32, (_M, _CH), 0)
    ciota = jax.lax.broadcasted_iota(jnp.int32, (_M, _CH), 1)
    x2 = jnp.sum(xT * xT, axis=0, keepdims=True)

    def sim_chunk(cent, c2, c):
        sl = slice(c * _CH, (c + 1) * _CH)
        xc = xT[:, sl]
        cx = jax.lax.dot_general(cent, xc, (((1,), (0,)), ((), ())),
                                 preferred_element_type=jnp.float32)
        return 2.0 * cx - c2 - x2[:, sl], xc

    x_ext = jnp.concatenate([xT, jnp.ones((1, n), jnp.float32)], axis=0)

    def km_iter(_, st):
        cent, num_pts, closest, kcounts, done = st
        c2h = 0.5 * jnp.sum(cent * cent, axis=1, keepdims=True)
        acc = jnp.zeros((_M, 4), jnp.float32)
        parts = []
        for c in range(nch):
            xc = xT[:, c * _CH:(c + 1) * _CH]
            simc = jax.lax.dot_general(
                cent, xc, (((1,), (0,)), ((), ())),
                preferred_element_type=jnp.float32) - c2h
            closc = jnp.argmax(simc, axis=0).reshape(1, _CH)
            maskc = (riota == closc).astype(jnp.float32)
            acc = acc + jax.lax.dot_general(
                maskc, x_ext[:, c * _CH:(c + 1) * _CH],
                (((1,), (1,)), ((), ())),
                preferred_element_type=jnp.float32)
            parts.append(closc)
        new_closest = jnp.concatenate(parts, axis=1)
        counts = acc[:, 3:4]
        c_grad = acc[:, 0:3] / (counts + 1e-8)
        error = jnp.sum((c_grad - cent) ** 2)
        lr = 0.9 / (num_pts + 1e-8) + 0.1
        cent_n = cent * (1.0 - lr) + c_grad * lr
        num_n = num_pts + counts
        cent = jnp.where(done, cent, cent_n)
        num_pts = jnp.where(done, num_pts, num_n)
        closest = jnp.where(done, closest, new_closest)
        kcounts = jnp.where(done, kcounts, counts)
        done = jnp.logical_or(done, error <= _TOL)
        return cent, num_pts, closest, kcounts, done

    st0 = (cent_s[...], jnp.ones((_M, 1), jnp.float32),
           jnp.zeros((1, n), jnp.int32), jnp.zeros((_M, 1), jnp.float32),
           jnp.zeros((), jnp.bool_))
    cent, _, closest, cnt, _ = jax.lax.fori_loop(0, _ITERS, km_iter, st0,
                                                 unroll=False)
    cent_s[...] = cent
    rpts_ref[0] = cent

    c2 = jnp.sum(cent * cent, axis=1, keepdims=True)
    cands = []
    for c in range(nch):
        sl = slice(c * _CH, (c + 1) * _CH)
        simc, _ = sim_chunk(cent, c2, c)
        member = riota == closest[:, sl]
        wc = jnp.where(member, simc, _NEG)
        work_s[:, sl] = wc
        v = jnp.max(wc, axis=1, keepdims=True)
        cands.append(v)
        for _ in range(_K - 1):
            v = jnp.max(jnp.where(wc < v, wc, -jnp.inf), axis=1,
                        keepdims=True)
            cands.append(v)
    cand = jnp.concatenate(cands, axis=1)
    candiota = jax.lax.broadcasted_iota(jnp.int32, cand.shape, 1)
    m0 = jnp.max(cand, axis=1, keepdims=True)
    t = m0
    for _ in range(_K - 1):
        am = jnp.argmax(cand, axis=1).reshape(_M, 1)
        cand = jnp.where(candiota == am, -jnp.inf, cand)
        t = jnp.max(cand, axis=1, keepdims=True)

    empty = cnt == 0.0
    total = jnp.zeros((_M, 1), jnp.float32)
    cf = jnp.zeros((_M, 64), jnp.float32)
    for c in range(nch):
        sl = slice(c * _CH, (c + 1) * _CH)
        wc = work_s[:, sl]
        w = jnp.where(wc >= t, jnp.exp(wc - m0), 0.0)
        if c == 0:
            w = jnp.where(empty, (ciota < _K).astype(jnp.float32), w)
        else:
            w = jnp.where(empty, 0.0, w)
        total = total + jnp.sum(w, axis=1, keepdims=True)
        cf = cf + jax.lax.dot_general(
            w, features_ref[0, :, sl], (((1,), (1,)), ((), ())),
            preferred_element_type=jnp.float32)
    rfeats_ref[0] = cf / total


@jax.jit
def kernel(points, features, centroids):
    nb, n, _ = points.shape
    nf = features.shape[1]
    points_t = jnp.swapaxes(points, 1, 2)
    rpts, rfeats_t = pl.pallas_call(
        _body,
        grid=(nb,),
        in_specs=[
            pl.BlockSpec((1, 3, n), lambda b: (b, 0, 0)),
            pl.BlockSpec((1, nf, n), lambda b: (b, 0, 0)),
            pl.BlockSpec((_M, 3), lambda b: (0, 0)),
        ],
        out_specs=[
            pl.BlockSpec((1, _M, 3), lambda b: (b, 0, 0)),
            pl.BlockSpec((1, _M, nf), lambda b: (b, 0, 0)),
        ],
        out_shape=[
            jax.ShapeDtypeStruct((nb, _M, 3), jnp.float32),
            jax.ShapeDtypeStruct((nb, _M, nf), jnp.float32),
        ],
        scratch_shapes=[
            pltpu.VMEM((_M, 3), jnp.float32),
            pltpu.VMEM((_M, n), jnp.float32),
        ],
        compiler_params=pltpu.CompilerParams(
            dimension_semantics=("arbitrary",)),
    )(points_t, features, centroids)
    r_feats = jnp.swapaxes(rfeats_t, 1, 2)
    return rpts, r_feats, rpts[nb - 1]
